# Initial kernel scaffold; baseline (speedup 1.0000x reference)
import jax
import jax.numpy as jnp
from jax.experimental import pallas as pl


def kernel(feat, kernel, src_ids, tgt_ids, feat_depth):
    raise NotImplementedError("write your pallas kernel here")


# speedup vs baseline: 6.7015x; 6.7015x over previous
import functools

import jax
import jax.numpy as jnp
from jax import lax
from jax.experimental import pallas as pl
from jax.experimental.pallas import tpu as pltpu
from jax.experimental.pallas import tpu_sc as plsc

N = 10000
C = 128
K = 27
E = 11852
E_TOT = K * E

NC = 2
NS = 16
NW = NC * NS
CB = 128
CHUNKS = -(-E_TOT // (NW * CB))
E_PAD = NW * CHUNKS * CB
ACC_ROWS = 10240
RPT = ACC_ROWS // NS


def _gemm_body(feat_ref, w_ref, y_ref):
    y_ref[0] = jnp.dot(feat_ref[...], w_ref[0],
                       preferred_element_type=jnp.float32)


def _node_gemm(feat, w):
    mb = 2000
    return pl.pallas_call(
        _gemm_body,
        grid=(K, N // mb),
        in_specs=[
            pl.BlockSpec((mb, C), lambda k, m: (m, 0)),
            pl.BlockSpec((1, C, C), lambda k, m: (k, 0, 0)),
        ],
        out_specs=pl.BlockSpec((1, mb, C), lambda k, m: (k, m, 0)),
        out_shape=jax.ShapeDtypeStruct((K, N, C), jnp.float32),
    )(feat, w)


def _scatter_body(y_hbm, src_hbm, tgt_hbm, zero_hbm, parts_hbm,
                  acc, src_v, tgt_v, rows_v, sem):
    c = lax.axis_index("c")
    s = lax.axis_index("s")
    wid = c * NS + s
    pltpu.sync_copy(zero_hbm.at[pl.ds(s * RPT, RPT)],
                    acc.at[pl.ds(s * RPT, RPT)])
    pltpu.sync_copy(src_hbm.at[wid], src_v)
    pltpu.sync_copy(tgt_hbm.at[wid], tgt_v)
    plsc.subcore_barrier()

    @pl.loop(0, CHUNKS)
    def _chunk(j):
        pltpu.async_copy(y_hbm.at[src_v.at[j]], rows_v, sem).wait()
        pltpu.sync_copy(rows_v, acc.at[tgt_v.at[j]], add=True)

    plsc.subcore_barrier()
    pltpu.sync_copy(acc.at[pl.ds(s * RPT, RPT)],
                    parts_hbm.at[c, pl.ds(s * RPT, RPT)])


_scatter_kernel = functools.partial(
    pl.kernel,
    out_type=jax.ShapeDtypeStruct((NC, ACC_ROWS, C), jnp.float32),
    mesh=plsc.VectorSubcoreMesh(core_axis_name="c", subcore_axis_name="s"),
    scratch_types=[
        pltpu.VMEM_SHARED((ACC_ROWS, C), jnp.float32),
        pltpu.VMEM((CHUNKS, CB), jnp.int32),
        pltpu.VMEM((CHUNKS, CB), jnp.int32),
        pltpu.VMEM((CB, C), jnp.float32),
        pltpu.SemaphoreType.DMA,
    ],
)(_scatter_body)


def _add_body(p0_ref, p1_ref, o_ref):
    o_ref[...] = p0_ref[0] + p1_ref[0]


def _sum_parts(parts):
    mb = 2000
    return pl.pallas_call(
        _add_body,
        grid=(N // mb,),
        in_specs=[
            pl.BlockSpec((1, mb, C), lambda m: (0, m, 0)),
            pl.BlockSpec((1, mb, C), lambda m: (1, m, 0)),
        ],
        out_specs=pl.BlockSpec((mb, C), lambda m: (m, 0)),
        out_shape=jax.ShapeDtypeStruct((N, C), jnp.float32),
    )(parts, parts)


def kernel(feat, kernel, src_ids, tgt_ids, feat_depth):
    src32 = (src_ids.astype(jnp.int32)
             + (jnp.arange(K, dtype=jnp.int32) * N)[:, None]).reshape(-1)
    tgt32 = tgt_ids.astype(jnp.int32).reshape(-1)
    pad = E_PAD - E_TOT
    src_w = jnp.concatenate(
        [src32, jnp.zeros((pad,), jnp.int32)]).reshape(NW, CHUNKS, CB)
    tgt_w = jnp.concatenate(
        [tgt32, jnp.full((pad,), N, jnp.int32)]).reshape(NW, CHUNKS, CB)

    y = _node_gemm(feat, kernel).reshape(K * N, C)
    zeros = jnp.zeros((ACC_ROWS, C), jnp.float32)
    parts = _scatter_kernel(y, src_w, tgt_w, zeros)
    out = _sum_parts(parts)
    return (out, feat_depth)
